# baseline (speedup 1.0000x reference)
import functools

import jax
import jax.numpy as jnp
from jax import lax
from jax.experimental import pallas as pl
from jax.experimental.pallas import tpu as pltpu
from jax.experimental.pallas import tpu_sc as plsc

N = 10000
E = 160000
IN_C = 256
D1, D2, D3 = 192, 128, 64

NC, NS, L = 2, 16, 16
NT = NC * NS
E_PAD = 161792
DEG_EPT = E_PAD // NT
AGG_EPT = E_PAD // NS
CHUNK = 128
NCHUNKS = AGG_EPT // CHUNK
ROWS_PT = N // NS
ACC_ROWS = N + L

BN = 1000
NB = N // BN


def _sc_mesh():
    return plsc.VectorSubcoreMesh(core_axis_name="c", subcore_axis_name="s")


_SC_PARAMS = pltpu.CompilerParams(needs_layout_passes=False,
                                  use_tc_tiling_on_sc=False)


_RED_W = 640
_RED_PAD = NS * _RED_W


@functools.cache
def _deg_kernel():
    @functools.partial(
        pl.kernel,
        out_type=jax.ShapeDtypeStruct((N,), jnp.float32),
        mesh=_sc_mesh(),
        compiler_params=_SC_PARAMS,
        scratch_types=[
            pltpu.VMEM((NCHUNKS, CHUNK), jnp.int32),
            pltpu.VMEM((ACC_ROWS,), jnp.float32),
            pltpu.VMEM((NS, _RED_W), jnp.float32),
            pltpu.VMEM((_RED_W,), jnp.float32),
            pltpu.VMEM_SHARED((NS, _RED_PAD), jnp.float32),
        ],
    )
    def deg_kernel(dst_hbm, out_hbm, didx, hist, rblk, rsum, spart):
        c = lax.axis_index("c")
        s = lax.axis_index("s")

        def zero_body(j, carry):
            hist[pl.ds(j * L, L)] = jnp.zeros((L,), jnp.float32)
            return carry

        lax.fori_loop(0, ACC_ROWS // L, zero_body, 0)
        pltpu.sync_copy(dst_hbm.at[s], didx)
        ones = jnp.ones((L,), jnp.float32)

        def body(j, carry):
            for k in range(CHUNK // L):
                idx = didx[j, pl.ds(k * L, L)]
                plsc.addupdate_scatter(hist, [idx], ones)
            return carry

        lax.fori_loop(0, NCHUNKS, body, 0)
        pltpu.sync_copy(hist.at[pl.ds(0, N)], spart.at[s, pl.ds(0, N)])
        plsc.subcore_barrier()
        base = s * _RED_W
        pltpu.sync_copy(spart.at[:, pl.ds(base, _RED_W)], rblk)
        for k in range(_RED_W // L):
            sl = pl.ds(k * L, L)
            acc16 = rblk[0, sl]
            for r in range(1, NS):
                acc16 = acc16 + rblk[r, sl]
            rsum[sl] = acc16
        half = N // NC
        lo = c * half

        @pl.when(jnp.logical_and(base >= lo, base < lo + half))
        def _():
            @pl.when(s < NS - 1)
            def _():
                pltpu.sync_copy(rsum, out_hbm.at[pl.ds(base, _RED_W)])

            @pl.when(s == NS - 1)
            def _():
                pltpu.sync_copy(rsum.at[pl.ds(0, N - (NS - 1) * _RED_W)],
                                out_hbm.at[pl.ds(base, N - (NS - 1) * _RED_W)])

    return deg_kernel


@functools.cache
def _agg_kernel(F2, depth, hbm_chunks):
    K = hbm_chunks
    scratch = [
        pltpu.VMEM_SHARED((ACC_ROWS, F2), jnp.float32),
        pltpu.VMEM((NCHUNKS, CHUNK), jnp.int32),
        pltpu.VMEM((depth, CHUNK, F2), jnp.float32),
        pltpu.SemaphoreType.DMA,
        pltpu.SemaphoreType.DMA,
        pltpu.SemaphoreType.DMA,
        pltpu.VMEM((max(K, 1), CHUNK), jnp.int32),
    ]
    if K < NCHUNKS:
        scratch.append(pltpu.VMEM((NCHUNKS - K, CHUNK), jnp.int32))
        scratch.append(pltpu.VMEM_SHARED((N, F2), jnp.float32))

    @functools.partial(
        pl.kernel,
        out_type=jax.ShapeDtypeStruct((NC, N, F2), jnp.float32),
        mesh=_sc_mesh(),
        compiler_params=_SC_PARAMS,
        scratch_types=scratch,
    )
    def agg_kernel(u_hbm, src_hbm, dst_hbm, out_hbm, acc, didx, rows,
                   gsem_h, gsem_s, ssem, sidx_h, *spmem_extra):
        c = lax.axis_index("c")
        s = lax.axis_index("s")
        row0 = s * ROWS_PT
        pltpu.sync_copy(u_hbm.at[pl.ds(c * N + row0, ROWS_PT)],
                        acc.at[pl.ds(row0, ROWS_PT)])
        if K > 0:
            pltpu.sync_copy(src_hbm.at[c, s, pl.ds(0, K)], sidx_h)
        if K < NCHUNKS:
            sidx_s, ucopy = spmem_extra
            pltpu.sync_copy(u_hbm.at[pl.ds(c * N + row0, ROWS_PT)],
                            ucopy.at[pl.ds(row0, ROWS_PT)])
            pltpu.sync_copy(src_hbm.at[0, s, pl.ds(K, NCHUNKS - K)], sidx_s)
        pltpu.sync_copy(dst_hbm.at[s], didx)
        plsc.subcore_barrier()
        ahead = depth - 2

        def issue_gather(jj, bb):
            if K >= NCHUNKS:
                pltpu.async_copy(u_hbm.at[sidx_h.at[jj]], rows.at[bb], gsem_h)
            else:
                @pl.when(jj < K)
                def _():
                    pltpu.async_copy(u_hbm.at[sidx_h.at[jj]], rows.at[bb],
                                     gsem_h)

                @pl.when(jj >= K)
                def _():
                    pltpu.async_copy(ucopy.at[sidx_s.at[jj - K]],
                                     rows.at[bb], gsem_s)

        def wait_gather(jj, bb):
            if K >= NCHUNKS:
                pltpu.make_async_copy(u_hbm.at[sidx_h.at[jj]], rows.at[bb],
                                      gsem_h).wait()
            else:
                @pl.when(jj < K)
                def _():
                    pltpu.make_async_copy(u_hbm.at[sidx_h.at[jj]],
                                          rows.at[bb], gsem_h).wait()

                @pl.when(jj >= K)
                def _():
                    pltpu.make_async_copy(ucopy.at[sidx_s.at[jj - K]],
                                          rows.at[bb], gsem_s).wait()

        assert K >= ahead
        for p in range(ahead):
            pltpu.async_copy(u_hbm.at[sidx_h.at[p]], rows.at[p], gsem_h)

        def body(j, carry):
            b = lax.rem(j, depth)

            @pl.when(j >= 2)
            def _():
                pltpu.make_async_copy(rows.at[b], acc.at[didx.at[j]],
                                      ssem).wait()

            @pl.when(j + ahead < NCHUNKS)
            def _():
                issue_gather(j + ahead, lax.rem(j + ahead, depth))

            wait_gather(j, b)
            pltpu.async_copy(rows.at[b], acc.at[didx.at[j]], ssem, add=True)
            return carry

        lax.fori_loop(0, NCHUNKS, body, 0)
        pltpu.make_async_copy(rows.at[0], acc.at[didx.at[0]], ssem).wait()
        pltpu.make_async_copy(rows.at[0], acc.at[didx.at[0]], ssem).wait()
        plsc.subcore_barrier()
        pltpu.sync_copy(acc.at[pl.ds(row0, ROWS_PT)],
                        out_hbm.at[c, pl.ds(row0, ROWS_PT)])

    return agg_kernel


def _mm_first(x, W, deg):
    H = W.shape[1] // 2
    Ws = [W[:, 0:64], W[:, H:H + 64], W[:, 64:H], W[:, H + 64:]]

    def body(x_ref, wa0, wa1, wb0, wb1, d_ref, oa_ref, ob_ref):
        xb = x_ref[...]
        d = lax.rsqrt(d_ref[...] + 1.0)
        oa_ref[0] = d * jnp.dot(xb, wa0[...], preferred_element_type=jnp.float32)
        oa_ref[1] = d * jnp.dot(xb, wa1[...], preferred_element_type=jnp.float32)
        ob_ref[0] = d * jnp.dot(xb, wb0[...], preferred_element_type=jnp.float32)
        ob_ref[1] = d * jnp.dot(xb, wb1[...], preferred_element_type=jnp.float32)

    return pl.pallas_call(
        body,
        grid=(NB,),
        in_specs=[
            pl.BlockSpec((BN, IN_C), lambda i: (i, 0)),
            pl.BlockSpec((IN_C, 64), lambda i: (0, 0)),
            pl.BlockSpec((IN_C, 64), lambda i: (0, 0)),
            pl.BlockSpec((IN_C, 32), lambda i: (0, 0)),
            pl.BlockSpec((IN_C, 32), lambda i: (0, 0)),
            pl.BlockSpec((BN, 1), lambda i: (i, 0)),
        ],
        out_specs=[
            pl.BlockSpec((NC, BN, 64), lambda i: (0, i, 0)),
            pl.BlockSpec((NC, BN, 32), lambda i: (0, i, 0)),
        ],
        out_shape=[
            jax.ShapeDtypeStruct((NC, N, 64), jnp.float32),
            jax.ShapeDtypeStruct((NC, N, 32), jnp.float32),
        ],
    )(x, *Ws, deg)


def _mm_mid(aggs, bprev, deg, W):
    Dprev = 2 * sum(a.shape[2] for a in aggs)
    F2 = W.shape[1] // 2
    Wa, Wb = W[:, :F2], W[:, F2:]
    npieces = len(aggs)

    def body(*refs):
        a_refs = refs[:2 * npieces]
        b_ref, d_ref, wa_ref, wb_ref, o_ref = refs[2 * npieces:]
        d = lax.rsqrt(d_ref[...] + 1.0)
        parts = []
        for c in range(NC):
            for g in range(npieces):
                parts.append(a_refs[c * npieces + g][0])
        h = jnp.concatenate(parts, axis=1)
        h = jnp.maximum(d * h + b_ref[...], 0.0)
        o_ref[0] = d * jnp.dot(h, wa_ref[...], preferred_element_type=jnp.float32)
        o_ref[1] = d * jnp.dot(h, wb_ref[...], preferred_element_type=jnp.float32)

    agg_specs = []
    agg_args = []
    for c in range(NC):
        for a in aggs:
            agg_specs.append(
                pl.BlockSpec((1, BN, a.shape[2]),
                             functools.partial(lambda cc, i: (cc, i, 0), c)))
            agg_args.append(a)
    return pl.pallas_call(
        body,
        grid=(NB,),
        in_specs=agg_specs + [
            pl.BlockSpec((Dprev,), lambda i: (0,)),
            pl.BlockSpec((BN, 1), lambda i: (i, 0)),
            pl.BlockSpec((Dprev, F2), lambda i: (0, 0)),
            pl.BlockSpec((Dprev, F2), lambda i: (0, 0)),
        ],
        out_specs=pl.BlockSpec((NC, BN, F2), lambda i: (0, i, 0)),
        out_shape=jax.ShapeDtypeStruct((NC, N, F2), jnp.float32),
    )(*agg_args, bprev, deg, Wa, Wb)


def _mm_last(agg, b3, deg):
    Fp2 = agg.shape[2]
    Dout = 2 * Fp2

    def body(a0_ref, a1_ref, b_ref, d_ref, o_ref):
        h = jnp.concatenate([a0_ref[0], a1_ref[0]], axis=1)
        o_ref[...] = lax.rsqrt(d_ref[...] + 1.0) * h + b_ref[...]

    return pl.pallas_call(
        body,
        grid=(NB,),
        in_specs=[
            pl.BlockSpec((1, BN, Fp2), lambda i: (0, i, 0)),
            pl.BlockSpec((1, BN, Fp2), lambda i: (1, i, 0)),
            pl.BlockSpec((Dout,), lambda i: (0,)),
            pl.BlockSpec((BN, 1), lambda i: (i, 0)),
        ],
        out_specs=pl.BlockSpec((BN, Dout), lambda i: (i, 0)),
        out_shape=jax.ShapeDtypeStruct((N, Dout), jnp.float32),
    )(agg, agg, b3, deg)


def kernel(x, edge_index, W1, b1, W2, b2, W3, b3):
    src = edge_index[0].astype(jnp.int32)
    dst = edge_index[1].astype(jnp.int32)
    pad = E_PAD - src.shape[0]
    src_p = jnp.concatenate([src, jnp.zeros((pad,), jnp.int32)])
    dst_p = jnp.concatenate([dst, jnp.full((pad,), N, jnp.int32)])
    src2 = jnp.stack([src_p, src_p + N]).reshape(NC, NS, NCHUNKS, CHUNK)
    dst3 = dst_p.reshape(NS, NCHUNKS, CHUNK)

    deg = _deg_kernel()(dst3).reshape(N, 1)

    u1a, u1b = _mm_first(x, W1, deg)
    a1a = _agg_kernel(64, 3, 32)(u1a.reshape(NC * N, 64), src2, dst3)
    a1b = _agg_kernel(32, 6, 32)(u1b.reshape(NC * N, 32), src2, dst3)
    u2 = _mm_mid([a1a, a1b], b1, deg, W2).reshape(NC * N, D2 // 2)
    a2 = _agg_kernel(D2 // 2, 3, 32)(u2, src2, dst3)
    u3 = _mm_mid([a2], b2, deg, W3).reshape(NC * N, D3 // 2)
    a3 = _agg_kernel(D3 // 2, 6, 32)(u3, src2, dst3)
    return _mm_last(a3, b3, deg)


# speedup vs baseline: 1.1243x; 1.1243x over previous
import functools

import jax
import jax.numpy as jnp
from jax import lax
from jax.experimental import pallas as pl
from jax.experimental.pallas import tpu as pltpu
from jax.experimental.pallas import tpu_sc as plsc

N = 10000
E = 160000
IN_C = 256
D1, D2, D3 = 192, 128, 64

NC, NS, L = 2, 16, 16
NT = NC * NS
E_PAD = 161792
DEG_EPT = E_PAD // NT
AGG_EPT = E_PAD // NS
CHUNK = 128
NCHUNKS = AGG_EPT // CHUNK
ROWS_PT = N // NS
ACC_ROWS = N + L

BN = 2000
NB = N // BN


def _sc_mesh():
    return plsc.VectorSubcoreMesh(core_axis_name="c", subcore_axis_name="s")


_SC_PARAMS = pltpu.CompilerParams(needs_layout_passes=False,
                                  use_tc_tiling_on_sc=False)


_RED_W = 640
_RED_PAD = NS * _RED_W


@functools.cache
def _deg_kernel():
    @functools.partial(
        pl.kernel,
        out_type=jax.ShapeDtypeStruct((N,), jnp.float32),
        mesh=_sc_mesh(),
        compiler_params=_SC_PARAMS,
        scratch_types=[
            pltpu.VMEM((NCHUNKS, CHUNK), jnp.int32),
            pltpu.VMEM((ACC_ROWS,), jnp.float32),
            pltpu.VMEM((NS, _RED_W), jnp.float32),
            pltpu.VMEM((_RED_W,), jnp.float32),
            pltpu.VMEM_SHARED((NS, _RED_PAD), jnp.float32),
        ],
    )
    def deg_kernel(dst_hbm, out_hbm, didx, hist, rblk, rsum, spart):
        c = lax.axis_index("c")
        s = lax.axis_index("s")

        def zero_body(j, carry):
            hist[pl.ds(j * L, L)] = jnp.zeros((L,), jnp.float32)
            return carry

        lax.fori_loop(0, ACC_ROWS // L, zero_body, 0)
        pltpu.sync_copy(dst_hbm.at[s], didx)
        ones = jnp.ones((L,), jnp.float32)

        def body(j, carry):
            for k in range(CHUNK // L):
                idx = didx[j, pl.ds(k * L, L)]
                plsc.addupdate_scatter(hist, [idx], ones)
            return carry

        lax.fori_loop(0, NCHUNKS, body, 0)
        pltpu.sync_copy(hist.at[pl.ds(0, N)], spart.at[s, pl.ds(0, N)])
        plsc.subcore_barrier()
        base = s * _RED_W
        pltpu.sync_copy(spart.at[:, pl.ds(base, _RED_W)], rblk)
        for k in range(_RED_W // L):
            sl = pl.ds(k * L, L)
            acc16 = rblk[0, sl]
            for r in range(1, NS):
                acc16 = acc16 + rblk[r, sl]
            rsum[sl] = acc16
        half = N // NC
        lo = c * half

        @pl.when(jnp.logical_and(base >= lo, base < lo + half))
        def _():
            @pl.when(s < NS - 1)
            def _():
                pltpu.sync_copy(rsum, out_hbm.at[pl.ds(base, _RED_W)])

            @pl.when(s == NS - 1)
            def _():
                pltpu.sync_copy(rsum.at[pl.ds(0, N - (NS - 1) * _RED_W)],
                                out_hbm.at[pl.ds(base, N - (NS - 1) * _RED_W)])

    return deg_kernel


@functools.cache
def _agg_kernel(F2, depth, hbm_chunks):
    K = hbm_chunks
    scratch = [
        pltpu.VMEM_SHARED((ACC_ROWS, F2), jnp.float32),
        pltpu.VMEM((NCHUNKS, CHUNK), jnp.int32),
        pltpu.VMEM((depth, CHUNK, F2), jnp.float32),
        pltpu.SemaphoreType.DMA,
        pltpu.SemaphoreType.DMA,
        pltpu.SemaphoreType.DMA,
        pltpu.VMEM((max(K, 1), CHUNK), jnp.int32),
    ]
    if K < NCHUNKS:
        scratch.append(pltpu.VMEM((NCHUNKS - K, CHUNK), jnp.int32))
        scratch.append(pltpu.VMEM_SHARED((N, F2), jnp.float32))

    @functools.partial(
        pl.kernel,
        out_type=jax.ShapeDtypeStruct((NC, N, F2), jnp.float32),
        mesh=_sc_mesh(),
        compiler_params=_SC_PARAMS,
        scratch_types=scratch,
    )
    def agg_kernel(u_hbm, src_hbm, dst_hbm, out_hbm, acc, didx, rows,
                   gsem_h, gsem_s, ssem, sidx_h, *spmem_extra):
        c = lax.axis_index("c")
        s = lax.axis_index("s")
        row0 = s * ROWS_PT
        pltpu.sync_copy(u_hbm.at[pl.ds(c * N + row0, ROWS_PT)],
                        acc.at[pl.ds(row0, ROWS_PT)])
        if K > 0:
            pltpu.sync_copy(src_hbm.at[c, s, pl.ds(0, K)], sidx_h)
        if K < NCHUNKS:
            sidx_s, ucopy = spmem_extra
            pltpu.sync_copy(u_hbm.at[pl.ds(c * N + row0, ROWS_PT)],
                            ucopy.at[pl.ds(row0, ROWS_PT)])
            pltpu.sync_copy(src_hbm.at[0, s, pl.ds(K, NCHUNKS - K)], sidx_s)
        pltpu.sync_copy(dst_hbm.at[s], didx)
        plsc.subcore_barrier()
        ahead = depth - 2

        def issue_gather(jj, bb):
            if K >= NCHUNKS:
                pltpu.async_copy(u_hbm.at[sidx_h.at[jj]], rows.at[bb], gsem_h)
            else:
                @pl.when(jj < K)
                def _():
                    pltpu.async_copy(u_hbm.at[sidx_h.at[jj]], rows.at[bb],
                                     gsem_h)

                @pl.when(jj >= K)
                def _():
                    pltpu.async_copy(ucopy.at[sidx_s.at[jj - K]],
                                     rows.at[bb], gsem_s)

        def wait_gather(jj, bb):
            if K >= NCHUNKS:
                pltpu.make_async_copy(u_hbm.at[sidx_h.at[jj]], rows.at[bb],
                                      gsem_h).wait()
            else:
                @pl.when(jj < K)
                def _():
                    pltpu.make_async_copy(u_hbm.at[sidx_h.at[jj]],
                                          rows.at[bb], gsem_h).wait()

                @pl.when(jj >= K)
                def _():
                    pltpu.make_async_copy(ucopy.at[sidx_s.at[jj - K]],
                                          rows.at[bb], gsem_s).wait()

        assert K >= ahead
        for p in range(ahead):
            pltpu.async_copy(u_hbm.at[sidx_h.at[p]], rows.at[p], gsem_h)

        def body(j, carry):
            b = lax.rem(j, depth)

            @pl.when(j >= 2)
            def _():
                pltpu.make_async_copy(rows.at[b], acc.at[didx.at[j]],
                                      ssem).wait()

            @pl.when(j + ahead < NCHUNKS)
            def _():
                issue_gather(j + ahead, lax.rem(j + ahead, depth))

            wait_gather(j, b)
            pltpu.async_copy(rows.at[b], acc.at[didx.at[j]], ssem, add=True)
            return carry

        lax.fori_loop(0, NCHUNKS, body, 0)
        pltpu.make_async_copy(rows.at[0], acc.at[didx.at[0]], ssem).wait()
        pltpu.make_async_copy(rows.at[0], acc.at[didx.at[0]], ssem).wait()
        plsc.subcore_barrier()
        pltpu.sync_copy(acc.at[pl.ds(row0, ROWS_PT)],
                        out_hbm.at[c, pl.ds(row0, ROWS_PT)])

    return agg_kernel


PB = BN // 2


def _mm_first(x, W, deg):
    H = W.shape[1] // 2
    Ws = [W[:, 0:64], W[:, H:H + 64], W[:, 64:H], W[:, H + 64:]]

    def body(x_ref, wa0, wa1, wb0, wb1, d_ref, oa_ref, ob_ref):
        xb = x_ref[...]
        d = lax.rsqrt(d_ref[...] + 1.0)
        oa_ref[0] = d * jnp.dot(xb, wa0[...], preferred_element_type=jnp.float32)
        oa_ref[1] = d * jnp.dot(xb, wa1[...], preferred_element_type=jnp.float32)
        ob_ref[0] = d * jnp.dot(xb, wb0[...], preferred_element_type=jnp.float32)
        ob_ref[1] = d * jnp.dot(xb, wb1[...], preferred_element_type=jnp.float32)

    return pl.pallas_call(
        body,
        grid=(NB,),
        in_specs=[
            pl.BlockSpec((BN, IN_C), lambda i: (i, 0)),
            pl.BlockSpec((IN_C, 64), lambda i: (0, 0)),
            pl.BlockSpec((IN_C, 64), lambda i: (0, 0)),
            pl.BlockSpec((IN_C, 32), lambda i: (0, 0)),
            pl.BlockSpec((IN_C, 32), lambda i: (0, 0)),
            pl.BlockSpec((BN, 1), lambda i: (i, 0)),
        ],
        out_specs=[
            pl.BlockSpec((NC, BN, 64), lambda i: (0, i, 0)),
            pl.BlockSpec((NC, BN, 32), lambda i: (0, i, 0)),
        ],
        out_shape=[
            jax.ShapeDtypeStruct((NC, N, 64), jnp.float32),
            jax.ShapeDtypeStruct((NC, N, 32), jnp.float32),
        ],
    )(x, *Ws, deg)


def _unpack_perm(p, w):
    return jnp.concatenate([p[:, :w], p[:, w:]], axis=0)


def _pack_perm(u):
    return jnp.concatenate([u[:PB], u[PB:]], axis=1)


def _mm2(a1ap, a1bp, bprev, degp, W):
    Wa, Wb = W[:, :64], W[:, 64:]

    def body(aa0, aa1, ab0, ab1, b_ref, dp_ref, wa, wb, o_ref):
        d = lax.rsqrt(_unpack_perm(dp_ref[...], 1) + 1.0)
        h = jnp.concatenate([
            _unpack_perm(aa0[...], 64), _unpack_perm(ab0[0], 32),
            _unpack_perm(aa1[...], 64), _unpack_perm(ab1[0], 32)], axis=1)
        h = jnp.maximum(d * h + b_ref[...], 0.0)
        o_ref[0] = _pack_perm(d * jnp.dot(h, wa[...], preferred_element_type=jnp.float32))
        o_ref[1] = _pack_perm(d * jnp.dot(h, wb[...], preferred_element_type=jnp.float32))

    return pl.pallas_call(
        body,
        grid=(NB,),
        in_specs=[
            pl.BlockSpec((PB, 128), lambda i: (i, 0)),
            pl.BlockSpec((PB, 128), lambda i: (NB + i, 0)),
            pl.BlockSpec((1, PB, 64), lambda i: (0, i, 0)),
            pl.BlockSpec((1, PB, 64), lambda i: (1, i, 0)),
            pl.BlockSpec((192,), lambda i: (0,)),
            pl.BlockSpec((PB, 2), lambda i: (i, 0)),
            pl.BlockSpec((192, 64), lambda i: (0, 0)),
            pl.BlockSpec((192, 64), lambda i: (0, 0)),
        ],
        out_specs=pl.BlockSpec((NC, PB, 128), lambda i: (0, i, 0)),
        out_shape=jax.ShapeDtypeStruct((NC, N // 2, 128), jnp.float32),
    )(a1ap, a1ap, a1bp, a1bp, bprev, degp, Wa, Wb)


def _mm3(a2p, bprev, degp, W):
    Wa, Wb = W[:, :32], W[:, 32:]

    def body(a0_ref, a1_ref, b_ref, dp_ref, wa_ref, wb_ref, o_ref):
        d = lax.rsqrt(_unpack_perm(dp_ref[...], 1) + 1.0)
        h = jnp.concatenate([_unpack_perm(a0_ref[...], 64),
                             _unpack_perm(a1_ref[...], 64)], axis=1)
        h = jnp.maximum(d * h + b_ref[...], 0.0)
        o_ref[0] = _pack_perm(d * jnp.dot(h, wa_ref[...], preferred_element_type=jnp.float32))
        o_ref[1] = _pack_perm(d * jnp.dot(h, wb_ref[...], preferred_element_type=jnp.float32))

    return pl.pallas_call(
        body,
        grid=(NB,),
        in_specs=[
            pl.BlockSpec((PB, 128), lambda i: (i, 0)),
            pl.BlockSpec((PB, 128), lambda i: (NB + i, 0)),
            pl.BlockSpec((128,), lambda i: (0,)),
            pl.BlockSpec((PB, 2), lambda i: (i, 0)),
            pl.BlockSpec((128, 32), lambda i: (0, 0)),
            pl.BlockSpec((128, 32), lambda i: (0, 0)),
        ],
        out_specs=pl.BlockSpec((NC, PB, 64), lambda i: (0, i, 0)),
        out_shape=jax.ShapeDtypeStruct((NC, N // 2, 64), jnp.float32),
    )(a2p, a2p, bprev, degp, Wa, Wb)


def _mm_last(agg, b3, deg):
    Fp2 = agg.shape[2]
    Dout = 2 * Fp2

    def body(a0_ref, a1_ref, b_ref, d_ref, o_ref):
        h = jnp.concatenate([a0_ref[0], a1_ref[0]], axis=1)
        o_ref[...] = lax.rsqrt(d_ref[...] + 1.0) * h + b_ref[...]

    return pl.pallas_call(
        body,
        grid=(NB,),
        in_specs=[
            pl.BlockSpec((1, BN, Fp2), lambda i: (0, i, 0)),
            pl.BlockSpec((1, BN, Fp2), lambda i: (1, i, 0)),
            pl.BlockSpec((Dout,), lambda i: (0,)),
            pl.BlockSpec((BN, 1), lambda i: (i, 0)),
        ],
        out_specs=pl.BlockSpec((BN, Dout), lambda i: (i, 0)),
        out_shape=jax.ShapeDtypeStruct((N, Dout), jnp.float32),
    )(agg, agg, b3, deg)


def kernel(x, edge_index, W1, b1, W2, b2, W3, b3):
    src = edge_index[0].astype(jnp.int32)
    dst = edge_index[1].astype(jnp.int32)
    pad = E_PAD - src.shape[0]
    src_p = jnp.concatenate([src, jnp.zeros((pad,), jnp.int32)])
    dst_p = jnp.concatenate([dst, jnp.full((pad,), N, jnp.int32)])
    src2 = jnp.stack([src_p, src_p + N]).reshape(NC, NS, NCHUNKS, CHUNK)
    dst3 = dst_p.reshape(NS, NCHUNKS, CHUNK)

    deg1 = _deg_kernel()(dst3)
    deg = deg1.reshape(N, 1)
    degp = deg1.reshape(N // 2, 2)

    u1a, u1b = _mm_first(x, W1, deg)
    a1a = _agg_kernel(64, 3, 32)(u1a.reshape(NC * N, 64), src2, dst3)
    a1b = _agg_kernel(32, 6, 32)(u1b.reshape(NC * N, 32), src2, dst3)
    u2p = _mm2(a1a.reshape(N, 128), a1b.reshape(NC, N // 2, 64), b1, degp, W2)
    a2 = _agg_kernel(64, 3, 32)(u2p.reshape(NC * N, 64), src2, dst3)
    u3p = _mm3(a2.reshape(N, 128), b2, degp, W3)
    a3 = _agg_kernel(32, 6, 32)(u3p.reshape(NC * N, 32), src2, dst3)
    return _mm_last(a3, b3, deg)
